# baseline (speedup 1.0000x reference)
import functools

import jax
import jax.numpy as jnp
from jax import lax
from jax.experimental import pallas as pl
from jax.experimental.pallas import tpu as pltpu
from jax.experimental.pallas import tpu_sc as plsc

_NBUF = 5
_DEPTH = 3


def _make_gather(N, V, D, num_cores, num_subcores):
    NW = num_cores * num_subcores
    n_per_w = N // NW
    C = 256
    n_chunks = n_per_w // C
    mesh = plsc.VectorSubcoreMesh(core_axis_name="c", subcore_axis_name="s")

    @functools.partial(
        pl.kernel,
        mesh=mesh,
        out_type=jax.ShapeDtypeStruct((N, D), jnp.float32),
        scratch_types=[
            pltpu.VMEM((n_per_w,), jnp.int32),
            pltpu.VMEM((_NBUF, C, D), jnp.float32),
            pltpu.SemaphoreType.DMA((_NBUF,)),
            pltpu.SemaphoreType.DMA((_NBUF,)),
        ],
        compiler_params=pltpu.CompilerParams(use_tc_tiling_on_sc=False),
    )
    def gather_kernel(idx_hbm, table_hbm, out_hbm, idx_v, rows_v, gsem, ssem):
        wid = lax.axis_index("s") * num_cores + lax.axis_index("c")
        base = wid * n_per_w

        pltpu.sync_copy(idx_hbm.at[pl.ds(base, n_per_w)], idx_v)

        def gather_copy(i, b):
            return pltpu.make_async_copy(
                table_hbm.at[idx_v.at[pl.ds(i * C, C)]], rows_v.at[b], gsem.at[b])

        def store_copy(i, b):
            return pltpu.make_async_copy(
                rows_v.at[b], out_hbm.at[pl.ds(base + i * C, C)], ssem.at[b])

        for j in range(_DEPTH):
            gather_copy(j, j).start()

        def body(i, carry):
            b = lax.rem(i, _NBUF)
            gather_copy(i, b).wait()
            store_copy(i, b).start()

            @pl.when(i + _DEPTH < n_chunks)
            def _next_gather():
                b2 = lax.rem(i + _DEPTH, _NBUF)

                @pl.when(i >= _NBUF - _DEPTH)
                def _recycle():
                    store_copy(i - (_NBUF - _DEPTH), b2).wait()

                gather_copy(i + _DEPTH, b2).start()

            return carry

        lax.fori_loop(0, n_chunks, body, 0)
        for j in range(n_chunks - _NBUF, n_chunks):
            store_copy(j, j % _NBUF).wait()

    return gather_kernel


def kernel(input_ids, weight):
    B, L = input_ids.shape
    V, D = weight.shape
    N = B * L
    info = plsc.get_sparse_core_info()
    flat_idx = input_ids.reshape(N).astype(jnp.int32)
    out = _make_gather(N, V, D, info.num_cores, info.num_subcores)(flat_idx, weight)
    return out.reshape(B, L, D)


# speedup vs baseline: 1.0039x; 1.0039x over previous
import functools

import jax
import jax.numpy as jnp
from jax import lax
from jax.experimental import pallas as pl
from jax.experimental.pallas import tpu as pltpu
from jax.experimental.pallas import tpu_sc as plsc

_NBUF = 5
_DEPTH = 3


def _make_gather(B, L, V, D, num_cores, num_subcores):
    NW = num_cores * num_subcores
    b_per_w = B // NW
    n_per_w = b_per_w * L
    mesh = plsc.VectorSubcoreMesh(core_axis_name="c", subcore_axis_name="s")

    @functools.partial(
        pl.kernel,
        mesh=mesh,
        out_type=jax.ShapeDtypeStruct((B, L, D), jnp.float32),
        scratch_types=[
            pltpu.VMEM((n_per_w,), jnp.int32),
            pltpu.VMEM((_NBUF, L, D), jnp.float32),
            pltpu.SemaphoreType.DMA((_NBUF,)),
            pltpu.SemaphoreType.DMA((_NBUF,)),
        ],
        compiler_params=pltpu.CompilerParams(use_tc_tiling_on_sc=False),
    )
    def gather_kernel(idx_hbm, table_hbm, out_hbm, idx_v, rows_v, gsem, ssem):
        wid = lax.axis_index("s") * num_cores + lax.axis_index("c")
        base = wid * n_per_w
        brow0 = wid * b_per_w

        pltpu.sync_copy(idx_hbm.at[pl.ds(base, n_per_w)], idx_v)

        def gather_copy(i, b):
            return pltpu.make_async_copy(
                table_hbm.at[idx_v.at[pl.ds(i * L, L)]], rows_v.at[b], gsem.at[b])

        def store_copy(i, b):
            return pltpu.make_async_copy(
                rows_v.at[b], out_hbm.at[brow0 + i], ssem.at[b])

        for j in range(_DEPTH):
            gather_copy(j, j).start()

        def body(i, carry):
            b = lax.rem(i, _NBUF)
            gather_copy(i, b).wait()
            store_copy(i, b).start()

            @pl.when(i + _DEPTH < b_per_w)
            def _next_gather():
                b2 = lax.rem(i + _DEPTH, _NBUF)

                @pl.when(i >= _NBUF - _DEPTH)
                def _recycle():
                    store_copy(i - (_NBUF - _DEPTH), b2).wait()

                gather_copy(i + _DEPTH, b2).start()

            return carry

        lax.fori_loop(0, b_per_w, body, 0)
        for j in range(b_per_w - _NBUF, b_per_w):
            store_copy(j, j % _NBUF).wait()

    return gather_kernel


def kernel(input_ids, weight):
    B, L = input_ids.shape
    V, D = weight.shape
    info = plsc.get_sparse_core_info()
    flat_idx = input_ids.reshape(B * L).astype(jnp.int32)
    return _make_gather(B, L, V, D, info.num_cores, info.num_subcores)(
        flat_idx, weight)
